# baseline (speedup 1.0000x reference)
import functools

import jax
import jax.numpy as jnp
from jax import lax
from jax.experimental import pallas as pl
from jax.experimental.pallas import tpu as pltpu
from jax.experimental.pallas import tpu_sc as plsc

NC = 2
NS = 16
CHUNK = 40
NSLOT = 5
LA = 3


def _edge_proj_body(ea_ref, we_ref, be_ref, o_ref):
    o_ref[...] = jnp.dot(ea_ref[...], we_ref[0],
                         preferred_element_type=jnp.float32) + be_ref[0]


def _edge_proj(edge_attr, We, be):
    E, DE = edge_attr.shape
    H = We.shape[1]
    HW = H // NC
    BE = 2000
    NEB = E // BE
    grid = (NC, NEB)
    return pl.pallas_call(
        _edge_proj_body,
        grid=grid,
        in_specs=[
            pl.BlockSpec((BE, DE), lambda q, i: (i, 0)),
            pl.BlockSpec((1, DE, HW), lambda q, i: (q, 0, 0)),
            pl.BlockSpec((1, 1, HW), lambda q, i: (q, 0, 0)),
        ],
        out_specs=pl.BlockSpec((BE, HW), lambda q, i: (q * NEB + i, 0)),
        out_shape=jax.ShapeDtypeStruct((NC * E, HW), jnp.float32),
    )(edge_attr,
      We.reshape(DE, NC, HW).transpose(1, 0, 2), be.reshape(NC, 1, HW))


def _sc_agg(xflat, ef, src, dst, zeros):
    N = zeros.shape[0]
    E = src.shape[0]
    HW = xflat.shape[1]
    epb = E // NS
    nch = epb // CHUNK
    nbody = nch // NSLOT
    rpt = (N // NS) // 8 * 8
    rem = N - rpt * NS
    mesh = plsc.VectorSubcoreMesh(core_axis_name="c", subcore_axis_name="s")

    src2 = jnp.concatenate([src, src + N])

    @functools.partial(
        pl.kernel,
        mesh=mesh,
        out_type=jax.ShapeDtypeStruct((NC * N, HW), jnp.float32),
        scratch_types=(
            [pltpu.VMEM((epb,), jnp.int32)]
            + [pltpu.VMEM((CHUNK,), jnp.int32) for _ in range(NSLOT)]
            + [pltpu.VMEM((CHUNK, HW), jnp.float32) for _ in range(NSLOT)]
            + [pltpu.VMEM_SHARED((N, HW), jnp.float32)]
            + [pltpu.SemaphoreType.DMA for _ in range(3 * NSLOT)]
        ),
    )
    def k(x_hbm, e_hbm, src_hbm, dst_hbm, z_hbm, out_hbm, *refs):
        sidx_all = refs[0]
        didx = refs[1:1 + NSLOT]
        ebufs = refs[1 + NSLOT:1 + 2 * NSLOT]
        agg = refs[1 + 2 * NSLOT]
        sems_e = refs[2 + 2 * NSLOT:2 + 3 * NSLOT]
        sems_g = refs[2 + 3 * NSLOT:2 + 4 * NSLOT]
        sems_s = refs[2 + 4 * NSLOT:2 + 5 * NSLOT]
        c = lax.axis_index("c")
        s = lax.axis_index("s")
        cN = c * N
        cE = c * E

        pltpu.sync_copy(z_hbm.at[pl.ds(s * rpt, rpt)],
                        agg.at[pl.ds(s * rpt, rpt)])
        if rem:
            @pl.when(s == 0)
            def _zero_rem():
                pltpu.sync_copy(z_hbm.at[pl.ds(rpt * NS, rem)],
                                agg.at[pl.ds(rpt * NS, rem)])
        pltpu.sync_copy(src_hbm.at[pl.ds(cE + s * epb, epb)], sidx_all)
        plsc.subcore_barrier()

        def loads(j, kk):
            pltpu.async_copy(e_hbm.at[pl.ds(cE + s * epb + kk * CHUNK, CHUNK)],
                             ebufs[j], sems_e[j])
            pltpu.async_copy(dst_hbm.at[pl.ds(s * epb + kk * CHUNK, CHUNK)],
                             didx[j], sems_e[j])

        def wait_loads(j):
            pltpu.make_async_copy(e_hbm.at[pl.ds(0, CHUNK)],
                                  ebufs[j], sems_e[j]).wait()
            pltpu.make_async_copy(dst_hbm.at[pl.ds(0, CHUNK)], didx[j],
                                  sems_e[j]).wait()

        def gather(j, kk):
            pltpu.async_copy(x_hbm.at[sidx_all.at[pl.ds(kk * CHUNK, CHUNK)]],
                             ebufs[j], sems_g[j], add=True)

        def wait_gather(j):
            pltpu.make_async_copy(x_hbm.at[pl.ds(0, CHUNK)],
                                  ebufs[j], sems_g[j]).wait()

        def scatter(j, kk):
            pltpu.async_copy(ebufs[j], agg.at[didx[j]],
                             sems_s[j], add=True)

        def wait_scatter(j):
            pltpu.make_async_copy(ebufs[j], out_hbm.at[pl.ds(0, CHUNK)],
                                  sems_s[j]).wait()

        def compute(j):
            eb = ebufs[j]

            def row(r, carry):
                for u in range(HW // 16):
                    sl = pl.ds(u * 16, 16)
                    eb[r, sl] = jnp.maximum(eb[r, sl], 0.0)
                return carry

            lax.fori_loop(0, CHUNK, row, 0)

        def chunk_step(j, kk, do_gather_next, do_wait_scatter, do_loads):
            if do_gather_next:
                wait_loads((j + 2) % NSLOT)
                gather((j + 2) % NSLOT, kk + 2)
            wait_gather(j)
            compute(j)
            scatter(j, kk)
            if do_wait_scatter:
                wait_scatter((j + LA) % NSLOT)
            if do_loads:
                loads((j + LA) % NSLOT, kk + LA)

        for j in range(LA):
            loads(j, j)
        for j in range(2):
            wait_loads(j)
            gather(j, j)
        for kk in range(NSLOT):
            chunk_step(kk % NSLOT, kk,
                       do_gather_next=kk + 2 < NSLOT + 2,
                       do_wait_scatter=kk >= 2,
                       do_loads=True)

        ilast = (nch - LA - NSLOT) // NSLOT

        def body(i, carry):
            for j in range(NSLOT):
                chunk_step(j, i * NSLOT + j, True, True, True)
            return carry

        lax.fori_loop(1, 1 + ilast, body, 0)

        for kk in range((1 + ilast) * NSLOT, nch):
            j = kk % NSLOT
            chunk_step(j, kk,
                       do_gather_next=kk + 2 < nch,
                       do_wait_scatter=kk + LA < nch,
                       do_loads=kk + LA < nch)
        for kk in range(nch - NSLOT, nch):
            wait_scatter(kk % NSLOT)

        plsc.subcore_barrier()
        pltpu.sync_copy(agg.at[pl.ds(s * rpt, rpt)],
                        out_hbm.at[pl.ds(cN + s * rpt, rpt)])
        if rem:
            @pl.when(s == 0)
            def _out_rem():
                pltpu.sync_copy(agg.at[pl.ds(rpt * NS, rem)],
                                out_hbm.at[pl.ds(cN + rpt * NS, rem)])

    return k(xflat, ef, src2, dst, zeros)


def _cat_input(xs_ref, ag_ref, eps_ref):
    f = 1.0 + eps_ref[0]
    return jnp.concatenate([f * xs_ref[q] + ag_ref[q] for q in range(NC)],
                           axis=1)


def _mlp_body(xs_ref, ag_ref, eps_ref, w1_ref, b1_ref, w2_ref, b2_ref, o_ref):
    a = _cat_input(xs_ref, ag_ref, eps_ref)
    t = jnp.maximum(jnp.dot(a.astype(jnp.bfloat16), w1_ref[...],
                            preferred_element_type=jnp.float32) + b1_ref[...],
                    0.0)
    o = jnp.maximum(jnp.dot(t.astype(jnp.bfloat16), w2_ref[...],
                            preferred_element_type=jnp.float32) + b2_ref[...],
                    0.0)
    HW = o.shape[1] // NC
    for q in range(NC):
        o_ref[q] = o[:, q * HW:(q + 1) * HW]


def _gin_mlp(xs, ag, eps, W1, b1, W2, b2):
    _, N, HW = xs.shape
    H = NC * HW
    BN = 2000
    grid = (N // BN,)
    return pl.pallas_call(
        _mlp_body,
        grid=grid,
        in_specs=[
            pl.BlockSpec((NC, BN, HW), lambda i: (0, i, 0)),
            pl.BlockSpec((NC, BN, HW), lambda i: (0, i, 0)),
            pl.BlockSpec(memory_space=pltpu.SMEM),
            pl.BlockSpec((H, H), lambda i: (0, 0)),
            pl.BlockSpec((1, H), lambda i: (0, 0)),
            pl.BlockSpec((H, H), lambda i: (0, 0)),
            pl.BlockSpec((1, H), lambda i: (0, 0)),
        ],
        out_specs=pl.BlockSpec((NC, BN, HW), lambda i: (0, i, 0)),
        out_shape=jax.ShapeDtypeStruct((NC, N, HW), jnp.float32),
    )(xs, ag, eps.reshape(1), W1.astype(jnp.bfloat16), b1.reshape(1, H),
      W2.astype(jnp.bfloat16), b2.reshape(1, H))


def _mlp_heads_body(xs_ref, ag_ref, eps_ref, w1_ref, b1_ref, w2_ref, b2_ref,
                    wmu_ref, bmu_ref, wlv_ref, blv_ref, mu_ref, lv_ref):
    a = _cat_input(xs_ref, ag_ref, eps_ref)
    t = jnp.maximum(jnp.dot(a.astype(jnp.bfloat16), w1_ref[...],
                            preferred_element_type=jnp.float32) + b1_ref[...],
                    0.0)
    h = jnp.maximum(jnp.dot(t.astype(jnp.bfloat16), w2_ref[...],
                            preferred_element_type=jnp.float32) + b2_ref[...],
                    0.0).astype(jnp.bfloat16)
    mu_ref[...] = jnp.dot(h, wmu_ref[...],
                          preferred_element_type=jnp.float32) + bmu_ref[...]
    lv_ref[...] = jnp.dot(h, wlv_ref[...],
                          preferred_element_type=jnp.float32) + blv_ref[...]


def _gin_mlp_heads(xs, ag, eps, W1, b1, W2, b2, Wmu, bmu, Wlv, blv):
    _, N, HW = xs.shape
    H = NC * HW
    L = Wmu.shape[1]
    BN = 2000
    grid = (N // BN,)
    return pl.pallas_call(
        _mlp_heads_body,
        grid=grid,
        in_specs=[
            pl.BlockSpec((NC, BN, HW), lambda i: (0, i, 0)),
            pl.BlockSpec((NC, BN, HW), lambda i: (0, i, 0)),
            pl.BlockSpec(memory_space=pltpu.SMEM),
            pl.BlockSpec((H, H), lambda i: (0, 0)),
            pl.BlockSpec((1, H), lambda i: (0, 0)),
            pl.BlockSpec((H, H), lambda i: (0, 0)),
            pl.BlockSpec((1, H), lambda i: (0, 0)),
            pl.BlockSpec((H, L), lambda i: (0, 0)),
            pl.BlockSpec((1, L), lambda i: (0, 0)),
            pl.BlockSpec((H, L), lambda i: (0, 0)),
            pl.BlockSpec((1, L), lambda i: (0, 0)),
        ],
        out_specs=[
            pl.BlockSpec((BN, L), lambda i: (i, 0)),
            pl.BlockSpec((BN, L), lambda i: (i, 0)),
        ],
        out_shape=[
            jax.ShapeDtypeStruct((N, L), jnp.float32),
            jax.ShapeDtypeStruct((N, L), jnp.float32),
        ],
    )(xs, ag, eps.reshape(1), W1.astype(jnp.bfloat16), b1.reshape(1, H),
      W2.astype(jnp.bfloat16), b2.reshape(1, H),
      Wmu.astype(jnp.bfloat16), bmu.reshape(1, L),
      Wlv.astype(jnp.bfloat16), blv.reshape(1, L))


def kernel(x, edge_index, edge_attr, We1, be1, eps1, W11, b11, W21, b21,
           We2, be2, eps2, W12, b12, W22, b22, Wmu, bmu, Wlv, blv):
    N, D = x.shape
    H = We1.shape[1]
    HW = H // NC
    src = edge_index[0].astype(jnp.int32)
    dst = edge_index[1].astype(jnp.int32)
    zeros = jnp.zeros((N, HW), jnp.float32)

    e1f = _edge_proj(edge_attr, We1, be1)

    x2 = jax.lax.optimization_barrier(x.reshape(N, NC, HW).transpose(1, 0, 2))
    agg1 = _sc_agg(x2.reshape(NC * N, HW), e1f, src, dst, zeros)
    e2f = _edge_proj(edge_attr, We2, be2)
    h1 = _gin_mlp(x2, agg1.reshape(NC, N, HW), eps1, W11, b11, W21, b21)

    agg2 = _sc_agg(h1.reshape(NC * N, HW), e2f, src, dst, zeros)
    mu, lv = _gin_mlp_heads(h1, agg2.reshape(NC, N, HW), eps2,
                            W12, b12, W22, b22, Wmu, bmu, Wlv, blv)
    return (mu, lv)


# speedup vs baseline: 1.1993x; 1.1993x over previous
import functools

import jax
import jax.numpy as jnp
from jax import lax
from jax.experimental import pallas as pl
from jax.experimental.pallas import tpu as pltpu
from jax.experimental.pallas import tpu_sc as plsc

NC = 2
NS = 16
CHUNK = 40
NSLOT = 5
LA = 3


def _edge_proj_body(ea_ref, we_ref, be_ref, o_ref):
    o_ref[...] = jnp.dot(ea_ref[...], we_ref[0],
                         preferred_element_type=jnp.float32) + be_ref[0]


def _edge_proj(edge_attr, We, be):
    E, DE = edge_attr.shape
    H = We.shape[1]
    HW = H // NC
    BE = 2000
    NEB = E // BE
    grid = (NC, NEB)
    return pl.pallas_call(
        _edge_proj_body,
        grid=grid,
        in_specs=[
            pl.BlockSpec((BE, DE), lambda q, i: (i, 0)),
            pl.BlockSpec((1, DE, HW), lambda q, i: (q, 0, 0)),
            pl.BlockSpec((1, 1, HW), lambda q, i: (q, 0, 0)),
        ],
        out_specs=pl.BlockSpec((BE, HW), lambda q, i: (q * NEB + i, 0)),
        out_shape=jax.ShapeDtypeStruct((NC * E, HW), jnp.float32),
    )(edge_attr,
      We.reshape(DE, NC, HW).transpose(1, 0, 2), be.reshape(NC, 1, HW))


def _sc_agg(xflat, ef, src, dst, zeros):
    N = zeros.shape[0]
    E = src.shape[0]
    HW = xflat.shape[1]
    epb = E // NS
    nch = epb // CHUNK
    nbody = nch // NSLOT
    rpt = (N // NS) // 8 * 8
    rem = N - rpt * NS
    mesh = plsc.VectorSubcoreMesh(core_axis_name="c", subcore_axis_name="s")

    src2 = jnp.concatenate([src, src + N])

    @functools.partial(
        pl.kernel,
        mesh=mesh,
        out_type=jax.ShapeDtypeStruct((NC * N, HW), jnp.float32),
        scratch_types=(
            [pltpu.VMEM((CHUNK,), jnp.int32) for _ in range(NSLOT)]
            + [pltpu.VMEM((CHUNK,), jnp.int32) for _ in range(NSLOT)]
            + [pltpu.VMEM((CHUNK, HW), jnp.float32) for _ in range(NSLOT)]
            + [pltpu.VMEM_SHARED((N, HW), jnp.float32)]
            + [pltpu.SemaphoreType.DMA for _ in range(3 * NSLOT)]
        ),
    )
    def k(x_hbm, e_hbm, src_hbm, dst_hbm, z_hbm, out_hbm, *refs):
        sidx = refs[0:NSLOT]
        didx = refs[NSLOT:2 * NSLOT]
        ebufs = refs[2 * NSLOT:3 * NSLOT]
        agg = refs[3 * NSLOT]
        sems_e = refs[3 * NSLOT + 1:3 * NSLOT + 1 + NSLOT]
        sems_g = refs[3 * NSLOT + 1 + NSLOT:3 * NSLOT + 1 + 2 * NSLOT]
        sems_s = refs[3 * NSLOT + 1 + 2 * NSLOT:3 * NSLOT + 1 + 3 * NSLOT]
        c = lax.axis_index("c")
        s = lax.axis_index("s")
        cN = c * N
        cE = c * E

        pltpu.sync_copy(z_hbm.at[pl.ds(s * rpt, rpt)],
                        agg.at[pl.ds(s * rpt, rpt)])
        if rem:
            @pl.when(s == 0)
            def _zero_rem():
                pltpu.sync_copy(z_hbm.at[pl.ds(rpt * NS, rem)],
                                agg.at[pl.ds(rpt * NS, rem)])
        plsc.subcore_barrier()

        def loads(j, kk):
            base = s * epb + kk * CHUNK
            pltpu.async_copy(e_hbm.at[pl.ds(cE + base, CHUNK)],
                             ebufs[j], sems_e[j])
            pltpu.async_copy(src_hbm.at[pl.ds(cE + base, CHUNK)],
                             sidx[j], sems_e[j])
            pltpu.async_copy(dst_hbm.at[pl.ds(base, CHUNK)],
                             didx[j], sems_e[j])

        def wait_loads(j):
            pltpu.make_async_copy(e_hbm.at[pl.ds(0, CHUNK)],
                                  ebufs[j], sems_e[j]).wait()
            pltpu.make_async_copy(src_hbm.at[pl.ds(0, CHUNK)], sidx[j],
                                  sems_e[j]).wait()
            pltpu.make_async_copy(dst_hbm.at[pl.ds(0, CHUNK)], didx[j],
                                  sems_e[j]).wait()

        def gather(j, kk):
            pltpu.async_copy(x_hbm.at[sidx[j]], ebufs[j], sems_g[j], add=True)

        def wait_gather(j):
            pltpu.make_async_copy(x_hbm.at[pl.ds(0, CHUNK)],
                                  ebufs[j], sems_g[j]).wait()

        def scatter(j, kk):
            pltpu.async_copy(ebufs[j], agg.at[didx[j]],
                             sems_s[j], add=True)

        def wait_scatter(j):
            pltpu.make_async_copy(ebufs[j], out_hbm.at[pl.ds(0, CHUNK)],
                                  sems_s[j]).wait()

        def compute(j):
            eb = ebufs[j]

            def row(r, carry):
                for u in range(HW // 16):
                    sl = pl.ds(u * 16, 16)
                    eb[r, sl] = jnp.maximum(eb[r, sl], 0.0)
                return carry

            lax.fori_loop(0, CHUNK, row, 0)

        def chunk_step(j, kk, do_gather_next, do_wait_scatter, do_loads):
            if do_gather_next:
                wait_loads((j + 1) % NSLOT)
                gather((j + 1) % NSLOT, kk + 1)
            wait_gather(j)
            compute(j)
            scatter(j, kk)
            if do_wait_scatter:
                wait_scatter((j + LA) % NSLOT)
            if do_loads:
                loads((j + LA) % NSLOT, kk + LA)

        for j in range(LA):
            loads(j, j)
        wait_loads(0)
        gather(0, 0)
        for kk in range(NSLOT):
            chunk_step(kk % NSLOT, kk,
                       do_gather_next=True,
                       do_wait_scatter=kk >= 2,
                       do_loads=True)

        ilast = (nch - LA - NSLOT) // NSLOT

        def body(i, carry):
            for j in range(NSLOT):
                chunk_step(j, i * NSLOT + j, True, True, True)
            return carry

        lax.fori_loop(1, 1 + ilast, body, 0)

        for kk in range((1 + ilast) * NSLOT, nch):
            j = kk % NSLOT
            chunk_step(j, kk,
                       do_gather_next=kk + 1 < nch,
                       do_wait_scatter=kk + LA < nch,
                       do_loads=kk + LA < nch)
        for kk in range(nch - NSLOT, nch):
            wait_scatter(kk % NSLOT)

        plsc.subcore_barrier()
        pltpu.sync_copy(agg.at[pl.ds(s * rpt, rpt)],
                        out_hbm.at[pl.ds(cN + s * rpt, rpt)])
        if rem:
            @pl.when(s == 0)
            def _out_rem():
                pltpu.sync_copy(agg.at[pl.ds(rpt * NS, rem)],
                                out_hbm.at[pl.ds(cN + rpt * NS, rem)])

    return k(xflat, ef, src2, dst, zeros)


def _cat_input(xs_ref, ag_ref, eps_ref):
    f = 1.0 + eps_ref[0]
    return jnp.concatenate([f * xs_ref[q] + ag_ref[q] for q in range(NC)],
                           axis=1)


def _mlp_body(xs_ref, ag_ref, eps_ref, w1_ref, b1_ref, w2_ref, b2_ref, o_ref):
    a = _cat_input(xs_ref, ag_ref, eps_ref)
    t = jnp.maximum(jnp.dot(a.astype(jnp.bfloat16), w1_ref[...],
                            preferred_element_type=jnp.float32) + b1_ref[...],
                    0.0)
    o = jnp.maximum(jnp.dot(t.astype(jnp.bfloat16), w2_ref[...],
                            preferred_element_type=jnp.float32) + b2_ref[...],
                    0.0)
    HW = o.shape[1] // NC
    for q in range(NC):
        o_ref[q] = o[:, q * HW:(q + 1) * HW]


def _gin_mlp(xs, ag, eps, W1, b1, W2, b2):
    _, N, HW = xs.shape
    H = NC * HW
    BN = 2000
    grid = (N // BN,)
    return pl.pallas_call(
        _mlp_body,
        grid=grid,
        in_specs=[
            pl.BlockSpec((NC, BN, HW), lambda i: (0, i, 0)),
            pl.BlockSpec((NC, BN, HW), lambda i: (0, i, 0)),
            pl.BlockSpec(memory_space=pltpu.SMEM),
            pl.BlockSpec((H, H), lambda i: (0, 0)),
            pl.BlockSpec((1, H), lambda i: (0, 0)),
            pl.BlockSpec((H, H), lambda i: (0, 0)),
            pl.BlockSpec((1, H), lambda i: (0, 0)),
        ],
        out_specs=pl.BlockSpec((NC, BN, HW), lambda i: (0, i, 0)),
        out_shape=jax.ShapeDtypeStruct((NC, N, HW), jnp.float32),
    )(xs, ag, eps.reshape(1), W1.astype(jnp.bfloat16), b1.reshape(1, H),
      W2.astype(jnp.bfloat16), b2.reshape(1, H))


def _mlp_heads_body(xs_ref, ag_ref, eps_ref, w1_ref, b1_ref, w2_ref, b2_ref,
                    wmu_ref, bmu_ref, wlv_ref, blv_ref, mu_ref, lv_ref):
    a = _cat_input(xs_ref, ag_ref, eps_ref)
    t = jnp.maximum(jnp.dot(a.astype(jnp.bfloat16), w1_ref[...],
                            preferred_element_type=jnp.float32) + b1_ref[...],
                    0.0)
    h = jnp.maximum(jnp.dot(t.astype(jnp.bfloat16), w2_ref[...],
                            preferred_element_type=jnp.float32) + b2_ref[...],
                    0.0).astype(jnp.bfloat16)
    mu_ref[...] = jnp.dot(h, wmu_ref[...],
                          preferred_element_type=jnp.float32) + bmu_ref[...]
    lv_ref[...] = jnp.dot(h, wlv_ref[...],
                          preferred_element_type=jnp.float32) + blv_ref[...]


def _gin_mlp_heads(xs, ag, eps, W1, b1, W2, b2, Wmu, bmu, Wlv, blv):
    _, N, HW = xs.shape
    H = NC * HW
    L = Wmu.shape[1]
    BN = 2000
    grid = (N // BN,)
    return pl.pallas_call(
        _mlp_heads_body,
        grid=grid,
        in_specs=[
            pl.BlockSpec((NC, BN, HW), lambda i: (0, i, 0)),
            pl.BlockSpec((NC, BN, HW), lambda i: (0, i, 0)),
            pl.BlockSpec(memory_space=pltpu.SMEM),
            pl.BlockSpec((H, H), lambda i: (0, 0)),
            pl.BlockSpec((1, H), lambda i: (0, 0)),
            pl.BlockSpec((H, H), lambda i: (0, 0)),
            pl.BlockSpec((1, H), lambda i: (0, 0)),
            pl.BlockSpec((H, L), lambda i: (0, 0)),
            pl.BlockSpec((1, L), lambda i: (0, 0)),
            pl.BlockSpec((H, L), lambda i: (0, 0)),
            pl.BlockSpec((1, L), lambda i: (0, 0)),
        ],
        out_specs=[
            pl.BlockSpec((BN, L), lambda i: (i, 0)),
            pl.BlockSpec((BN, L), lambda i: (i, 0)),
        ],
        out_shape=[
            jax.ShapeDtypeStruct((N, L), jnp.float32),
            jax.ShapeDtypeStruct((N, L), jnp.float32),
        ],
    )(xs, ag, eps.reshape(1), W1.astype(jnp.bfloat16), b1.reshape(1, H),
      W2.astype(jnp.bfloat16), b2.reshape(1, H),
      Wmu.astype(jnp.bfloat16), bmu.reshape(1, L),
      Wlv.astype(jnp.bfloat16), blv.reshape(1, L))


def kernel(x, edge_index, edge_attr, We1, be1, eps1, W11, b11, W21, b21,
           We2, be2, eps2, W12, b12, W22, b22, Wmu, bmu, Wlv, blv):
    N, D = x.shape
    H = We1.shape[1]
    HW = H // NC
    src = edge_index[0].astype(jnp.int32)
    dst = edge_index[1].astype(jnp.int32)
    zeros = jnp.zeros((N, HW), jnp.float32)

    e1f = _edge_proj(edge_attr, We1, be1)

    x2 = jax.lax.optimization_barrier(x.reshape(N, NC, HW).transpose(1, 0, 2))
    agg1 = _sc_agg(x2.reshape(NC * N, HW), e1f, src, dst, zeros)
    e2f = _edge_proj(edge_attr, We2, be2)
    h1 = _gin_mlp(x2, agg1.reshape(NC, N, HW), eps1, W11, b11, W21, b21)

    agg2 = _sc_agg(h1.reshape(NC * N, HW), e2f, src, dst, zeros)
    mu, lv = _gin_mlp_heads(h1, agg2.reshape(NC, N, HW), eps2,
                            W12, b12, W22, b22, Wmu, bmu, Wlv, blv)
    return (mu, lv)


# speedup vs baseline: 1.1997x; 1.0004x over previous
import functools

import jax
import jax.numpy as jnp
from jax import lax
from jax.experimental import pallas as pl
from jax.experimental.pallas import tpu as pltpu
from jax.experimental.pallas import tpu_sc as plsc

NC = 2
NS = 16
CHUNK = 40
NSLOT = 6
LA = 3


def _edge_proj_body(ea_ref, we_ref, be_ref, o_ref):
    o_ref[...] = jnp.dot(ea_ref[...], we_ref[0],
                         preferred_element_type=jnp.float32) + be_ref[0]


def _edge_proj(edge_attr, We, be):
    E, DE = edge_attr.shape
    H = We.shape[1]
    HW = H // NC
    BE = 2000
    NEB = E // BE
    grid = (NC, NEB)
    return pl.pallas_call(
        _edge_proj_body,
        grid=grid,
        in_specs=[
            pl.BlockSpec((BE, DE), lambda q, i: (i, 0)),
            pl.BlockSpec((1, DE, HW), lambda q, i: (q, 0, 0)),
            pl.BlockSpec((1, 1, HW), lambda q, i: (q, 0, 0)),
        ],
        out_specs=pl.BlockSpec((BE, HW), lambda q, i: (q * NEB + i, 0)),
        out_shape=jax.ShapeDtypeStruct((NC * E, HW), jnp.float32),
    )(edge_attr,
      We.reshape(DE, NC, HW).transpose(1, 0, 2), be.reshape(NC, 1, HW))


def _sc_agg(xflat, ef, src, dst, zeros):
    N = zeros.shape[0]
    E = src.shape[0]
    HW = xflat.shape[1]
    epb = E // NS
    nch = epb // CHUNK
    nbody = nch // NSLOT
    rpt = (N // NS) // 8 * 8
    rem = N - rpt * NS
    mesh = plsc.VectorSubcoreMesh(core_axis_name="c", subcore_axis_name="s")

    src2 = jnp.concatenate([src, src + N])

    @functools.partial(
        pl.kernel,
        mesh=mesh,
        out_type=jax.ShapeDtypeStruct((NC * N, HW), jnp.float32),
        scratch_types=(
            [pltpu.VMEM((CHUNK,), jnp.int32) for _ in range(NSLOT)]
            + [pltpu.VMEM((CHUNK,), jnp.int32) for _ in range(NSLOT)]
            + [pltpu.VMEM((CHUNK, HW), jnp.float32) for _ in range(NSLOT)]
            + [pltpu.VMEM_SHARED((N, HW), jnp.float32)]
            + [pltpu.SemaphoreType.DMA for _ in range(3 * NSLOT)]
        ),
    )
    def k(x_hbm, e_hbm, src_hbm, dst_hbm, z_hbm, out_hbm, *refs):
        sidx = refs[0:NSLOT]
        didx = refs[NSLOT:2 * NSLOT]
        ebufs = refs[2 * NSLOT:3 * NSLOT]
        agg = refs[3 * NSLOT]
        sems_e = refs[3 * NSLOT + 1:3 * NSLOT + 1 + NSLOT]
        sems_g = refs[3 * NSLOT + 1 + NSLOT:3 * NSLOT + 1 + 2 * NSLOT]
        sems_s = refs[3 * NSLOT + 1 + 2 * NSLOT:3 * NSLOT + 1 + 3 * NSLOT]
        c = lax.axis_index("c")
        s = lax.axis_index("s")
        cN = c * N
        cE = c * E

        pltpu.sync_copy(z_hbm.at[pl.ds(s * rpt, rpt)],
                        agg.at[pl.ds(s * rpt, rpt)])
        if rem:
            @pl.when(s == 0)
            def _zero_rem():
                pltpu.sync_copy(z_hbm.at[pl.ds(rpt * NS, rem)],
                                agg.at[pl.ds(rpt * NS, rem)])
        plsc.subcore_barrier()

        def loads(j, kk):
            base = s * epb + kk * CHUNK
            pltpu.async_copy(e_hbm.at[pl.ds(cE + base, CHUNK)],
                             ebufs[j], sems_e[j])
            pltpu.async_copy(src_hbm.at[pl.ds(cE + base, CHUNK)],
                             sidx[j], sems_e[j])
            pltpu.async_copy(dst_hbm.at[pl.ds(base, CHUNK)],
                             didx[j], sems_e[j])

        def wait_loads(j):
            pltpu.make_async_copy(e_hbm.at[pl.ds(0, CHUNK)],
                                  ebufs[j], sems_e[j]).wait()
            pltpu.make_async_copy(src_hbm.at[pl.ds(0, CHUNK)], sidx[j],
                                  sems_e[j]).wait()
            pltpu.make_async_copy(dst_hbm.at[pl.ds(0, CHUNK)], didx[j],
                                  sems_e[j]).wait()

        def gather(j, kk):
            pltpu.async_copy(x_hbm.at[sidx[j]], ebufs[j], sems_g[j], add=True)

        def wait_gather(j):
            pltpu.make_async_copy(x_hbm.at[pl.ds(0, CHUNK)],
                                  ebufs[j], sems_g[j]).wait()

        def scatter(j, kk):
            pltpu.async_copy(ebufs[j], agg.at[didx[j]],
                             sems_s[j], add=True)

        def wait_scatter(j):
            pltpu.make_async_copy(ebufs[j], out_hbm.at[pl.ds(0, CHUNK)],
                                  sems_s[j]).wait()

        def compute(j):
            eb = ebufs[j]

            def row(r, carry):
                for u in range(HW // 16):
                    sl = pl.ds(u * 16, 16)
                    eb[r, sl] = jnp.maximum(eb[r, sl], 0.0)
                return carry

            lax.fori_loop(0, CHUNK, row, 0)

        def chunk_step(j, kk, do_gather_next, do_wait_scatter, do_loads):
            if do_gather_next:
                wait_loads((j + 1) % NSLOT)
                gather((j + 1) % NSLOT, kk + 1)
            wait_gather(j)
            compute(j)
            scatter(j, kk)
            if do_wait_scatter:
                wait_scatter((j + LA) % NSLOT)
            if do_loads:
                loads((j + LA) % NSLOT, kk + LA)

        for j in range(LA):
            loads(j, j)
        wait_loads(0)
        gather(0, 0)
        for kk in range(NSLOT):
            chunk_step(kk % NSLOT, kk,
                       do_gather_next=True,
                       do_wait_scatter=kk >= NSLOT - LA,
                       do_loads=True)

        ilast = (nch - LA - NSLOT) // NSLOT

        def body(i, carry):
            for j in range(NSLOT):
                chunk_step(j, i * NSLOT + j, True, True, True)
            return carry

        lax.fori_loop(1, 1 + ilast, body, 0)

        for kk in range((1 + ilast) * NSLOT, nch):
            j = kk % NSLOT
            chunk_step(j, kk,
                       do_gather_next=kk + 1 < nch,
                       do_wait_scatter=kk + LA < nch,
                       do_loads=kk + LA < nch)
        for kk in range(nch - NSLOT, nch):
            wait_scatter(kk % NSLOT)

        plsc.subcore_barrier()
        pltpu.sync_copy(agg.at[pl.ds(s * rpt, rpt)],
                        out_hbm.at[pl.ds(cN + s * rpt, rpt)])
        if rem:
            @pl.when(s == 0)
            def _out_rem():
                pltpu.sync_copy(agg.at[pl.ds(rpt * NS, rem)],
                                out_hbm.at[pl.ds(cN + rpt * NS, rem)])

    return k(xflat, ef, src2, dst, zeros)


def _cat_input(xs_ref, ag_ref, eps_ref):
    f = 1.0 + eps_ref[0]
    return jnp.concatenate([f * xs_ref[q] + ag_ref[q] for q in range(NC)],
                           axis=1)


def _mlp_body(xs_ref, ag_ref, eps_ref, w1_ref, b1_ref, w2_ref, b2_ref, o_ref):
    a = _cat_input(xs_ref, ag_ref, eps_ref)
    t = jnp.maximum(jnp.dot(a.astype(jnp.bfloat16), w1_ref[...],
                            preferred_element_type=jnp.float32) + b1_ref[...],
                    0.0)
    o = jnp.maximum(jnp.dot(t.astype(jnp.bfloat16), w2_ref[...],
                            preferred_element_type=jnp.float32) + b2_ref[...],
                    0.0)
    HW = o.shape[1] // NC
    for q in range(NC):
        o_ref[q] = o[:, q * HW:(q + 1) * HW]


def _gin_mlp(xs, ag, eps, W1, b1, W2, b2):
    _, N, HW = xs.shape
    H = NC * HW
    BN = 2000
    grid = (N // BN,)
    return pl.pallas_call(
        _mlp_body,
        grid=grid,
        in_specs=[
            pl.BlockSpec((NC, BN, HW), lambda i: (0, i, 0)),
            pl.BlockSpec((NC, BN, HW), lambda i: (0, i, 0)),
            pl.BlockSpec(memory_space=pltpu.SMEM),
            pl.BlockSpec((H, H), lambda i: (0, 0)),
            pl.BlockSpec((1, H), lambda i: (0, 0)),
            pl.BlockSpec((H, H), lambda i: (0, 0)),
            pl.BlockSpec((1, H), lambda i: (0, 0)),
        ],
        out_specs=pl.BlockSpec((NC, BN, HW), lambda i: (0, i, 0)),
        out_shape=jax.ShapeDtypeStruct((NC, N, HW), jnp.float32),
    )(xs, ag, eps.reshape(1), W1.astype(jnp.bfloat16), b1.reshape(1, H),
      W2.astype(jnp.bfloat16), b2.reshape(1, H))


def _mlp_heads_body(xs_ref, ag_ref, eps_ref, w1_ref, b1_ref, w2_ref, b2_ref,
                    wmu_ref, bmu_ref, wlv_ref, blv_ref, mu_ref, lv_ref):
    a = _cat_input(xs_ref, ag_ref, eps_ref)
    t = jnp.maximum(jnp.dot(a.astype(jnp.bfloat16), w1_ref[...],
                            preferred_element_type=jnp.float32) + b1_ref[...],
                    0.0)
    h = jnp.maximum(jnp.dot(t.astype(jnp.bfloat16), w2_ref[...],
                            preferred_element_type=jnp.float32) + b2_ref[...],
                    0.0).astype(jnp.bfloat16)
    mu_ref[...] = jnp.dot(h, wmu_ref[...],
                          preferred_element_type=jnp.float32) + bmu_ref[...]
    lv_ref[...] = jnp.dot(h, wlv_ref[...],
                          preferred_element_type=jnp.float32) + blv_ref[...]


def _gin_mlp_heads(xs, ag, eps, W1, b1, W2, b2, Wmu, bmu, Wlv, blv):
    _, N, HW = xs.shape
    H = NC * HW
    L = Wmu.shape[1]
    BN = 2000
    grid = (N // BN,)
    return pl.pallas_call(
        _mlp_heads_body,
        grid=grid,
        in_specs=[
            pl.BlockSpec((NC, BN, HW), lambda i: (0, i, 0)),
            pl.BlockSpec((NC, BN, HW), lambda i: (0, i, 0)),
            pl.BlockSpec(memory_space=pltpu.SMEM),
            pl.BlockSpec((H, H), lambda i: (0, 0)),
            pl.BlockSpec((1, H), lambda i: (0, 0)),
            pl.BlockSpec((H, H), lambda i: (0, 0)),
            pl.BlockSpec((1, H), lambda i: (0, 0)),
            pl.BlockSpec((H, L), lambda i: (0, 0)),
            pl.BlockSpec((1, L), lambda i: (0, 0)),
            pl.BlockSpec((H, L), lambda i: (0, 0)),
            pl.BlockSpec((1, L), lambda i: (0, 0)),
        ],
        out_specs=[
            pl.BlockSpec((BN, L), lambda i: (i, 0)),
            pl.BlockSpec((BN, L), lambda i: (i, 0)),
        ],
        out_shape=[
            jax.ShapeDtypeStruct((N, L), jnp.float32),
            jax.ShapeDtypeStruct((N, L), jnp.float32),
        ],
    )(xs, ag, eps.reshape(1), W1.astype(jnp.bfloat16), b1.reshape(1, H),
      W2.astype(jnp.bfloat16), b2.reshape(1, H),
      Wmu.astype(jnp.bfloat16), bmu.reshape(1, L),
      Wlv.astype(jnp.bfloat16), blv.reshape(1, L))


def kernel(x, edge_index, edge_attr, We1, be1, eps1, W11, b11, W21, b21,
           We2, be2, eps2, W12, b12, W22, b22, Wmu, bmu, Wlv, blv):
    N, D = x.shape
    H = We1.shape[1]
    HW = H // NC
    src = edge_index[0].astype(jnp.int32)
    dst = edge_index[1].astype(jnp.int32)
    zeros = jnp.zeros((N, HW), jnp.float32)

    e1f = _edge_proj(edge_attr, We1, be1)

    x2 = jax.lax.optimization_barrier(x.reshape(N, NC, HW).transpose(1, 0, 2))
    agg1 = _sc_agg(x2.reshape(NC * N, HW), e1f, src, dst, zeros)
    e2f = _edge_proj(edge_attr, We2, be2)
    h1 = _gin_mlp(x2, agg1.reshape(NC, N, HW), eps1, W11, b11, W21, b21)

    agg2 = _sc_agg(h1.reshape(NC * N, HW), e2f, src, dst, zeros)
    mu, lv = _gin_mlp_heads(h1, agg2.reshape(NC, N, HW), eps2,
                            W12, b12, W22, b22, Wmu, bmu, Wlv, blv)
    return (mu, lv)


# speedup vs baseline: 1.2026x; 1.0024x over previous
import functools

import jax
import jax.numpy as jnp
from jax import lax
from jax.experimental import pallas as pl
from jax.experimental.pallas import tpu as pltpu
from jax.experimental.pallas import tpu_sc as plsc

NC = 2
NS = 16
CHUNK = 40
NSLOT = 5
LA = 3


def _edge_proj_body(ea_ref, we_ref, be_ref, o_ref):
    o_ref[...] = jnp.dot(ea_ref[...], we_ref[0],
                         preferred_element_type=jnp.float32) + be_ref[0]


def _edge_proj(edge_attr, We, be):
    E, DE = edge_attr.shape
    H = We.shape[1]
    HW = H // NC
    BE = 2000
    NEB = E // BE
    grid = (NC, NEB)
    return pl.pallas_call(
        _edge_proj_body,
        grid=grid,
        in_specs=[
            pl.BlockSpec((BE, DE), lambda q, i: (i, 0)),
            pl.BlockSpec((1, DE, HW), lambda q, i: (q, 0, 0)),
            pl.BlockSpec((1, 1, HW), lambda q, i: (q, 0, 0)),
        ],
        out_specs=pl.BlockSpec((BE, HW), lambda q, i: (q * NEB + i, 0)),
        out_shape=jax.ShapeDtypeStruct((NC * E, HW), jnp.float32),
    )(edge_attr,
      We.reshape(DE, NC, HW).transpose(1, 0, 2), be.reshape(NC, 1, HW))


def _sc_agg(xflat, ef, src, dst, zeros):
    N = zeros.shape[0]
    E = src.shape[0]
    HW = xflat.shape[1]
    epb = E // NS
    nch = epb // CHUNK
    nbody = nch // NSLOT
    rpt = (N // NS) // 8 * 8
    rem = N - rpt * NS
    mesh = plsc.VectorSubcoreMesh(core_axis_name="c", subcore_axis_name="s")

    src2 = jnp.concatenate([src, src + N])

    @functools.partial(
        pl.kernel,
        mesh=mesh,
        out_type=jax.ShapeDtypeStruct((NC * N, HW), jnp.float32),
        scratch_types=(
            [pltpu.VMEM((CHUNK,), jnp.int32) for _ in range(NSLOT)]
            + [pltpu.VMEM((CHUNK,), jnp.int32) for _ in range(NSLOT)]
            + [pltpu.VMEM((CHUNK, HW), jnp.float32) for _ in range(NSLOT)]
            + [pltpu.VMEM_SHARED((N, HW), jnp.float32)]
            + [pltpu.SemaphoreType.DMA for _ in range(3 * NSLOT)]
        ),
    )
    def k(x_hbm, e_hbm, src_hbm, dst_hbm, z_hbm, out_hbm, *refs):
        sidx = refs[0:NSLOT]
        didx = refs[NSLOT:2 * NSLOT]
        ebufs = refs[2 * NSLOT:3 * NSLOT]
        agg = refs[3 * NSLOT]
        sems_e = refs[3 * NSLOT + 1:3 * NSLOT + 1 + NSLOT]
        sems_g = refs[3 * NSLOT + 1 + NSLOT:3 * NSLOT + 1 + 2 * NSLOT]
        sems_s = refs[3 * NSLOT + 1 + 2 * NSLOT:3 * NSLOT + 1 + 3 * NSLOT]
        c = lax.axis_index("c")
        s = lax.axis_index("s")
        cN = c * N
        cE = c * E

        pltpu.sync_copy(z_hbm.at[pl.ds(s * rpt, rpt)],
                        agg.at[pl.ds(s * rpt, rpt)])
        if rem:
            @pl.when(s == 0)
            def _zero_rem():
                pltpu.sync_copy(z_hbm.at[pl.ds(rpt * NS, rem)],
                                agg.at[pl.ds(rpt * NS, rem)])
        plsc.subcore_barrier()

        def loads(j, kk):
            base = s * epb + kk * CHUNK
            pltpu.async_copy(e_hbm.at[pl.ds(cE + base, CHUNK)],
                             ebufs[j], sems_e[j])
            pltpu.async_copy(src_hbm.at[pl.ds(cE + base, CHUNK)],
                             sidx[j], sems_e[j])
            pltpu.async_copy(dst_hbm.at[pl.ds(base, CHUNK)],
                             didx[j], sems_e[j])

        def wait_loads(j):
            pltpu.make_async_copy(e_hbm.at[pl.ds(0, CHUNK)],
                                  ebufs[j], sems_e[j]).wait()
            pltpu.make_async_copy(src_hbm.at[pl.ds(0, CHUNK)], sidx[j],
                                  sems_e[j]).wait()
            pltpu.make_async_copy(dst_hbm.at[pl.ds(0, CHUNK)], didx[j],
                                  sems_e[j]).wait()

        def gather(j, kk):
            pltpu.async_copy(x_hbm.at[sidx[j]], ebufs[j], sems_g[j], add=True)

        def wait_gather(j):
            pltpu.make_async_copy(x_hbm.at[pl.ds(0, CHUNK)],
                                  ebufs[j], sems_g[j]).wait()

        def scatter(j, kk):
            pltpu.async_copy(ebufs[j], agg.at[didx[j]],
                             sems_s[j], add=True)

        def wait_scatter(j):
            pltpu.make_async_copy(ebufs[j], out_hbm.at[pl.ds(0, CHUNK)],
                                  sems_s[j]).wait()

        def compute(j):
            eb = ebufs[j]

            def row(r, carry):
                for u in range(HW // 16):
                    sl = pl.ds(u * 16, 16)
                    eb[r, sl] = jnp.maximum(eb[r, sl], 0.0)
                return carry

            lax.fori_loop(0, CHUNK, row, 0)

        def chunk_step(j, kk, do_gather_next, do_wait_scatter, do_loads):
            if do_gather_next:
                wait_loads((j + 1) % NSLOT)
                gather((j + 1) % NSLOT, kk + 1)
            wait_gather(j)
            compute(j)
            scatter(j, kk)
            if do_wait_scatter:
                wait_scatter((j + LA) % NSLOT)
            if do_loads:
                loads((j + LA) % NSLOT, kk + LA)

        for j in range(LA):
            loads(j, j)
        wait_loads(0)
        gather(0, 0)
        for kk in range(NSLOT):
            chunk_step(kk % NSLOT, kk,
                       do_gather_next=True,
                       do_wait_scatter=kk >= 2,
                       do_loads=True)

        ilast = (nch - LA - NSLOT) // NSLOT

        def body(i, carry):
            for j in range(NSLOT):
                chunk_step(j, i * NSLOT + j, True, True, True)
            return carry

        lax.fori_loop(1, 1 + ilast, body, 0)

        for kk in range((1 + ilast) * NSLOT, nch):
            j = kk % NSLOT
            chunk_step(j, kk,
                       do_gather_next=kk + 1 < nch,
                       do_wait_scatter=kk + LA < nch,
                       do_loads=kk + LA < nch)
        for kk in range(nch - NSLOT, nch):
            wait_scatter(kk % NSLOT)

        plsc.subcore_barrier()
        pltpu.sync_copy(agg.at[pl.ds(s * rpt, rpt)],
                        out_hbm.at[pl.ds(cN + s * rpt, rpt)])
        if rem:
            @pl.when(s == 0)
            def _out_rem():
                pltpu.sync_copy(agg.at[pl.ds(rpt * NS, rem)],
                                out_hbm.at[pl.ds(cN + rpt * NS, rem)])

    return k(xflat, ef, src2, dst, zeros)


def _cat_input(xs_ref, ag_ref, eps_ref):
    f = 1.0 + eps_ref[0]
    return jnp.concatenate([f * xs_ref[q] + ag_ref[q] for q in range(NC)],
                           axis=1)


def _mlp_body(xs_ref, ag_ref, eps_ref, w1_ref, b1_ref, w2_ref, b2_ref, o_ref):
    a = _cat_input(xs_ref, ag_ref, eps_ref)
    t = jnp.maximum(jnp.dot(a.astype(jnp.bfloat16), w1_ref[...],
                            preferred_element_type=jnp.float32) + b1_ref[...],
                    0.0)
    o = jnp.maximum(jnp.dot(t.astype(jnp.bfloat16), w2_ref[...],
                            preferred_element_type=jnp.float32) + b2_ref[...],
                    0.0)
    HW = o.shape[1] // NC
    for q in range(NC):
        o_ref[q] = o[:, q * HW:(q + 1) * HW]


def _gin_mlp(xs, ag, eps, W1, b1, W2, b2):
    _, N, HW = xs.shape
    H = NC * HW
    BN = 2000
    grid = (N // BN,)
    return pl.pallas_call(
        _mlp_body,
        grid=grid,
        in_specs=[
            pl.BlockSpec((NC, BN, HW), lambda i: (0, i, 0)),
            pl.BlockSpec((NC, BN, HW), lambda i: (0, i, 0)),
            pl.BlockSpec(memory_space=pltpu.SMEM),
            pl.BlockSpec((H, H), lambda i: (0, 0)),
            pl.BlockSpec((1, H), lambda i: (0, 0)),
            pl.BlockSpec((H, H), lambda i: (0, 0)),
            pl.BlockSpec((1, H), lambda i: (0, 0)),
        ],
        out_specs=pl.BlockSpec((NC, BN, HW), lambda i: (0, i, 0)),
        out_shape=jax.ShapeDtypeStruct((NC, N, HW), jnp.float32),
    )(xs, ag, eps.reshape(1), W1.astype(jnp.bfloat16), b1.reshape(1, H),
      W2.astype(jnp.bfloat16), b2.reshape(1, H))


def _mlp_heads_body(xs_ref, ag_ref, eps_ref, w1_ref, b1_ref, w2_ref, b2_ref,
                    wmu_ref, bmu_ref, wlv_ref, blv_ref, mu_ref, lv_ref):
    a = _cat_input(xs_ref, ag_ref, eps_ref)
    t = jnp.maximum(jnp.dot(a.astype(jnp.bfloat16), w1_ref[...],
                            preferred_element_type=jnp.float32) + b1_ref[...],
                    0.0)
    h = jnp.maximum(jnp.dot(t.astype(jnp.bfloat16), w2_ref[...],
                            preferred_element_type=jnp.float32) + b2_ref[...],
                    0.0).astype(jnp.bfloat16)
    mu_ref[...] = jnp.dot(h, wmu_ref[...],
                          preferred_element_type=jnp.float32) + bmu_ref[...]
    lv_ref[...] = jnp.dot(h, wlv_ref[...],
                          preferred_element_type=jnp.float32) + blv_ref[...]


def _gin_mlp_heads(xs, ag, eps, W1, b1, W2, b2, Wmu, bmu, Wlv, blv):
    _, N, HW = xs.shape
    H = NC * HW
    L = Wmu.shape[1]
    BN = 2000
    grid = (N // BN,)
    return pl.pallas_call(
        _mlp_heads_body,
        grid=grid,
        in_specs=[
            pl.BlockSpec((NC, BN, HW), lambda i: (0, i, 0)),
            pl.BlockSpec((NC, BN, HW), lambda i: (0, i, 0)),
            pl.BlockSpec(memory_space=pltpu.SMEM),
            pl.BlockSpec((H, H), lambda i: (0, 0)),
            pl.BlockSpec((1, H), lambda i: (0, 0)),
            pl.BlockSpec((H, H), lambda i: (0, 0)),
            pl.BlockSpec((1, H), lambda i: (0, 0)),
            pl.BlockSpec((H, L), lambda i: (0, 0)),
            pl.BlockSpec((1, L), lambda i: (0, 0)),
            pl.BlockSpec((H, L), lambda i: (0, 0)),
            pl.BlockSpec((1, L), lambda i: (0, 0)),
        ],
        out_specs=[
            pl.BlockSpec((BN, L), lambda i: (i, 0)),
            pl.BlockSpec((BN, L), lambda i: (i, 0)),
        ],
        out_shape=[
            jax.ShapeDtypeStruct((N, L), jnp.float32),
            jax.ShapeDtypeStruct((N, L), jnp.float32),
        ],
    )(xs, ag, eps.reshape(1), W1.astype(jnp.bfloat16), b1.reshape(1, H),
      W2.astype(jnp.bfloat16), b2.reshape(1, H),
      Wmu.astype(jnp.bfloat16), bmu.reshape(1, L),
      Wlv.astype(jnp.bfloat16), blv.reshape(1, L))


def kernel(x, edge_index, edge_attr, We1, be1, eps1, W11, b11, W21, b21,
           We2, be2, eps2, W12, b12, W22, b22, Wmu, bmu, Wlv, blv):
    N, D = x.shape
    H = We1.shape[1]
    HW = H // NC
    src = edge_index[0].astype(jnp.int32)
    dst = edge_index[1].astype(jnp.int32)
    zeros = jnp.zeros((N, HW), jnp.float32)

    e1f = _edge_proj(edge_attr, We1, be1)

    x2 = jax.lax.optimization_barrier(x.reshape(N, NC, HW).transpose(1, 0, 2))
    agg1 = _sc_agg(x2.reshape(NC * N, HW), e1f, src, dst, zeros)
    e2f = _edge_proj(edge_attr, We2, be2)
    h1 = _gin_mlp(x2, agg1.reshape(NC, N, HW), eps1, W11, b11, W21, b21)

    agg2 = _sc_agg(h1.reshape(NC * N, HW), e2f, src, dst, zeros)
    mu, lv = _gin_mlp_heads(h1, agg2.reshape(NC, N, HW), eps2,
                            W12, b12, W22, b22, Wmu, bmu, Wlv, blv)
    return (mu, lv)
